# baseline (speedup 1.0000x reference)
import jax
import jax.numpy as jnp
from jax import lax
from jax.experimental import pallas as pl
from jax.experimental.pallas import tpu as pltpu

_B = 128
_G = 8
_U = 8
_THRESH = 0.5
_NEG = -3e38


def _score_tile(sst, sen, spk, sht, cst, cen, cpk, cht, car):
    inter_start = jnp.maximum(cst, sst)
    inter_end = jnp.minimum(cen, sen)
    inter_len = jnp.maximum(inter_end - inter_start, 0.0)
    inter_h = jnp.minimum(cht, sht)
    inter_area = inter_len * inter_h
    sar = (sen - sst) * sht
    union_area = sar + car - inter_area
    iou = inter_area / union_area
    pd = jnp.abs(spk - cpk)
    union_start = jnp.minimum(cst, sst)
    union_end = jnp.maximum(cen, sen)
    ud = union_end - union_start
    return iou - pd / ud


def _nms_body(bst_ref, ben_ref, bpk_ref, bht_ref,
              rst_ref, ren_ref, rpk_ref, rht_ref,
              out_ref, pst_ref, pen_ref, ppk_ref, pht_ref, cur_ref):
    k = pl.program_id(0)
    B = _B
    G = _G

    @pl.when(k == 0)
    def _init():
        zero = jnp.zeros(pst_ref.shape, jnp.float32)
        pst_ref[...] = zero
        pen_ref[...] = zero
        ppk_ref[...] = zero
        pht_ref[...] = zero
        cur_ref[0] = 0

    cst = rst_ref[0]
    cen = ren_ref[0]
    cpk = rpk_ref[0]
    cht = rht_ref[0]
    car = (cen - cst) * cht

    cur = cur_ref[0]
    neg = jnp.full((G, B), _NEG, jnp.float32)

    def group(o):
        def g(base):
            return _score_tile(pst_ref[pl.ds(base, G), :],
                               pen_ref[pl.ds(base, G), :],
                               ppk_ref[pl.ds(base, G), :],
                               pht_ref[pl.ds(base, G), :],
                               cst, cen, cpk, cht, car)
        return g(o)

    def cross(t, accs):
        a0, a1 = accs
        base = _U * G * t
        for i in range(0, _U, 2):
            a0 = jnp.maximum(a0, group(base + i * G))
            a1 = jnp.maximum(a1, group(base + (i + 1) * G))
        return (a0, a1)

    a0, a1 = lax.fori_loop(0, cur // (_U * G), cross, (neg, neg))
    mx = jnp.max(jnp.maximum(a0, a1), axis=0, keepdims=True)
    ext = jnp.where(mx > _THRESH, 0.0, 1.0)

    lane = lax.broadcasted_iota(jnp.int32, (B, B), 1)
    sub = lax.broadcasted_iota(jnp.int32, (B, B), 0)
    ut = (sub <= lane).astype(jnp.float32)
    lane8 = lax.broadcasted_iota(jnp.int32, (G, B), 1)
    sub8 = lax.broadcasted_iota(jnp.int32, (G, B), 0)
    mats = []
    for i in range(B // G):
        s_i = _score_tile(bst_ref[0, pl.ds(i * G, G), :],
                          ben_ref[0, pl.ds(i * G, G), :],
                          bpk_ref[0, pl.ds(i * G, G), :],
                          bht_ref[0, pl.ds(i * G, G), :],
                          cst, cen, cpk, cht, car)
        tri = (sub8 + (i * G)) < lane8
        mats.append(jnp.where((s_i > _THRESH) & tri, 1.0, 0.0))
    mfull = jnp.concatenate(mats, axis=0)

    def fp_cond(c):
        return c[1] > 0

    def fstep(krow):
        cnt = lax.dot_general(krow, mfull, (((1,), (0,)), ((), ())),
                              preferred_element_type=jnp.float32)
        return jnp.where(cnt > 0.5, 0.0, ext)

    def fp_body(c):
        krow, _ = c
        new = krow
        for _i in range(4):
            new = fstep(new)
        changed = jnp.sum(jnp.abs(new - krow)).astype(jnp.int32)
        return new, changed

    krow, _ = lax.while_loop(fp_cond, fp_body, (ext, jnp.int32(1)))

    out_ref[0] = krow

    n_k = jnp.sum(krow).astype(jnp.int32)
    pos = lax.dot_general(krow, ut, (((1,), (0,)), ((), ())),
                          preferred_element_type=jnp.float32)
    pos_i = pos.astype(jnp.int32) - 1
    perm = ((sub == pos_i) & (krow > 0.5)).astype(jnp.float32)

    sub4 = lax.broadcasted_iota(jnp.int32, (4, B), 0)
    x4 = (jnp.where(sub4 == 0, cst, 0.0) + jnp.where(sub4 == 1, cen, 0.0)
          + jnp.where(sub4 == 2, cpk, 0.0) + jnp.where(sub4 == 3, cht, 0.0))
    chunk = lax.dot_general(perm, x4, (((1,), (1,)), ((), ())),
                            preferred_element_type=jnp.float32,
                            precision=lax.Precision.HIGHEST)

    def put(plane_ref, col):
        plane_ref[pl.ds(cur, B), :] = lax.broadcast_in_dim(col, (B, B), (0, 1))

    put(pst_ref, chunk[:, 0:1])
    put(pen_ref, chunk[:, 1:2])
    put(ppk_ref, chunk[:, 2:3])
    put(pht_ref, chunk[:, 3:4])
    align = _U * G
    cur_ref[0] = cur + ((n_k + align - 1) // align) * align


def _nms_keep(reps, rows, interpret=False):
    K = reps[0].shape[0]
    cap = K * _B + _B
    pspec = pl.BlockSpec((1, _B, _B), lambda k: (k, 0, 0))
    rspec = pl.BlockSpec((1, 1, _B), lambda k: (k, 0, 0))
    keep = pl.pallas_call(
        _nms_body,
        grid=(K,),
        in_specs=[pspec] * 4 + [rspec] * 4,
        out_specs=rspec,
        out_shape=jax.ShapeDtypeStruct((K, 1, _B), jnp.float32),
        scratch_shapes=[pltpu.VMEM((cap, _B), jnp.float32)] * 4
        + [pltpu.SMEM((1,), jnp.int32)],
        interpret=interpret,
    )(*reps, *rows)
    return keep.reshape(K * _B)


def _run(output, interpret=False):
    n, c = output.shape
    order = jnp.argsort(-output[:, 0])
    boxes = output[order]
    K = -(-n // _B)
    npad = K * _B
    pad = jnp.zeros((npad - n, c), jnp.float32)
    pad = pad.at[:, 1].set(1e9).at[:, 2].set(2e9).at[:, 3].set(1.5e9)
    pad = pad.at[:, 4].set(1.0)
    bp = jnp.concatenate([boxes, pad], axis=0)
    st = bp[:, 1]
    en = bp[:, 2]
    pk = bp[:, 3]
    ht = bp[:, 4]
    reps = [jnp.broadcast_to(v.reshape(K, _B, 1), (K, _B, _B))
            for v in (st, en, pk, ht)]
    rows = [v.reshape(K, 1, _B) for v in (st, en, pk, ht)]
    keep = _nms_keep(reps, rows, interpret=interpret)[:n]
    return boxes[:, 1:] * keep[:, None]


def kernel(output):
    return _run(output)


# speedup vs baseline: 1.1069x; 1.1069x over previous
import jax
import jax.numpy as jnp
from jax import lax
from jax.experimental import pallas as pl
from jax.experimental.pallas import tpu as pltpu

_B = 128
_G = 8
_U = 8
_THRESH = 0.5
_NEG = -3e38


def _score_tile(sst, sen, spk, sht, cst, cen, cpk, cht, car):
    inter_start = jnp.maximum(cst, sst)
    inter_end = jnp.minimum(cen, sen)
    inter_len = jnp.maximum(inter_end - inter_start, 0.0)
    inter_h = jnp.minimum(cht, sht)
    inter_area = inter_len * inter_h
    sar = (sen - sst) * sht
    union_area = sar + car - inter_area
    iou = inter_area / union_area
    pd = jnp.abs(spk - cpk)
    union_start = jnp.minimum(cst, sst)
    union_end = jnp.maximum(cen, sen)
    ud = union_end - union_start
    return iou - pd / ud


def _nms_body(bst_ref, ben_ref, bpk_ref, bht_ref,
              rst_ref, ren_ref, rpk_ref, rht_ref,
              out_ref, pst_ref, pen_ref, ppk_ref, pht_ref, cur_ref):
    k = pl.program_id(0)
    B = _B
    G = _G

    @pl.when(k == 0)
    def _init():
        zero = jnp.zeros(pst_ref.shape, jnp.float32)
        pst_ref[...] = zero
        pen_ref[...] = zero
        ppk_ref[...] = zero
        pht_ref[...] = zero
        cur_ref[0] = 0

    cst = rst_ref[0]
    cen = ren_ref[0]
    cpk = rpk_ref[0]
    cht = rht_ref[0]
    car = (cen - cst) * cht

    cur = cur_ref[0]
    neg = jnp.full((G, B), _NEG, jnp.float32)

    def group(o):
        def g(base):
            return _score_tile(pst_ref[pl.ds(base, G), :],
                               pen_ref[pl.ds(base, G), :],
                               ppk_ref[pl.ds(base, G), :],
                               pht_ref[pl.ds(base, G), :],
                               cst, cen, cpk, cht, car)
        return g(o)

    def cross(t, accs):
        a0, a1 = accs
        base = _U * G * t
        for i in range(0, _U, 2):
            a0 = jnp.maximum(a0, group(base + i * G))
            a1 = jnp.maximum(a1, group(base + (i + 1) * G))
        return (a0, a1)

    a0, a1 = lax.fori_loop(0, cur // (_U * G), cross, (neg, neg))
    mx = jnp.max(jnp.maximum(a0, a1), axis=0, keepdims=True)
    ext = jnp.where(mx > _THRESH, 0.0, 1.0)

    lane = lax.broadcasted_iota(jnp.int32, (B, B), 1)
    sub = lax.broadcasted_iota(jnp.int32, (B, B), 0)
    ut = (sub <= lane).astype(jnp.float32)
    lane8 = lax.broadcasted_iota(jnp.int32, (G, B), 1)
    sub8 = lax.broadcasted_iota(jnp.int32, (G, B), 0)
    mats = []
    for i in range(B // G):
        s_i = _score_tile(bst_ref[0, pl.ds(i * G, G), :],
                          ben_ref[0, pl.ds(i * G, G), :],
                          bpk_ref[0, pl.ds(i * G, G), :],
                          bht_ref[0, pl.ds(i * G, G), :],
                          cst, cen, cpk, cht, car)
        tri = (sub8 + (i * G)) < lane8
        mats.append(jnp.where((s_i > _THRESH) & tri, 1.0, 0.0))
    mfull = jnp.concatenate(mats, axis=0)

    def fp_cond(c):
        return c[1] > 0

    def fstep(krow):
        cnt = lax.dot_general(krow, mfull, (((1,), (0,)), ((), ())),
                              preferred_element_type=jnp.float32)
        return jnp.where(cnt > 0.5, 0.0, ext)

    def fp_body(c):
        krow, _ = c
        new = fstep(krow)
        changed = jnp.sum(jnp.abs(new - krow)).astype(jnp.int32)
        return new, changed

    krow, _ = lax.while_loop(fp_cond, fp_body, (ext, jnp.int32(1)))

    out_ref[0] = krow

    n_k = jnp.sum(krow).astype(jnp.int32)
    pos = lax.dot_general(krow, ut, (((1,), (0,)), ((), ())),
                          preferred_element_type=jnp.float32)
    pos_i = pos.astype(jnp.int32) - 1
    perm = ((sub == pos_i) & (krow > 0.5)).astype(jnp.float32)

    sub4 = lax.broadcasted_iota(jnp.int32, (4, B), 0)
    x4 = (jnp.where(sub4 == 0, cst, 0.0) + jnp.where(sub4 == 1, cen, 0.0)
          + jnp.where(sub4 == 2, cpk, 0.0) + jnp.where(sub4 == 3, cht, 0.0))
    chunk = lax.dot_general(perm, x4, (((1,), (1,)), ((), ())),
                            preferred_element_type=jnp.float32,
                            precision=lax.Precision.HIGHEST)

    def put(plane_ref, col):
        plane_ref[pl.ds(cur, B), :] = lax.broadcast_in_dim(col, (B, B), (0, 1))

    put(pst_ref, chunk[:, 0:1])
    put(pen_ref, chunk[:, 1:2])
    put(ppk_ref, chunk[:, 2:3])
    put(pht_ref, chunk[:, 3:4])
    align = _U * G
    cur_ref[0] = cur + ((n_k + align - 1) // align) * align


def _nms_keep(reps, rows, interpret=False):
    K = reps[0].shape[0]
    cap = K * _B + _B
    pspec = pl.BlockSpec((1, _B, _B), lambda k: (k, 0, 0))
    rspec = pl.BlockSpec((1, 1, _B), lambda k: (k, 0, 0))
    keep = pl.pallas_call(
        _nms_body,
        grid=(K,),
        in_specs=[pspec] * 4 + [rspec] * 4,
        out_specs=rspec,
        out_shape=jax.ShapeDtypeStruct((K, 1, _B), jnp.float32),
        scratch_shapes=[pltpu.VMEM((cap, _B), jnp.float32)] * 4
        + [pltpu.SMEM((1,), jnp.int32)],
        interpret=interpret,
    )(*reps, *rows)
    return keep.reshape(K * _B)


def _run(output, interpret=False):
    n, c = output.shape
    order = jnp.argsort(-output[:, 0])
    boxes = output[order]
    K = -(-n // _B)
    npad = K * _B
    pad = jnp.zeros((npad - n, c), jnp.float32)
    pad = pad.at[:, 1].set(1e9).at[:, 2].set(2e9).at[:, 3].set(1.5e9)
    pad = pad.at[:, 4].set(1.0)
    bp = jnp.concatenate([boxes, pad], axis=0)
    st = bp[:, 1]
    en = bp[:, 2]
    pk = bp[:, 3]
    ht = bp[:, 4]
    reps = [jnp.broadcast_to(v.reshape(K, _B, 1), (K, _B, _B))
            for v in (st, en, pk, ht)]
    rows = [v.reshape(K, 1, _B) for v in (st, en, pk, ht)]
    keep = _nms_keep(reps, rows, interpret=interpret)[:n]
    return boxes[:, 1:] * keep[:, None]


def kernel(output):
    return _run(output)
